# baseline (speedup 1.0000x reference)
import jax
import jax.numpy as jnp
from jax import lax
from jax.experimental import pallas as pl
from jax.experimental.pallas import tpu as pltpu
from jax.experimental.pallas import tpu_sc as plsc

VOCAB = 1000000
D = 64
DP = 128
BATCH = 4096
SEQ = 200
N = BATCH * SEQ
L = 16

NC = 2
NS = 16
NW = NC * NS
B_PER_W = BATCH // NW


def _gather_body(xT_hbm, tpad_hbm, out_hbm, idx_v, g0, g1, o0, o1,
                 gs0, gs1, ss0, ss1):
    G = [g0, g1]
    O = [o0, o1]
    gs = [gs0, gs1]
    ss = [ss0, ss1]
    wid = lax.axis_index("s") * NC + lax.axis_index("c")
    col0 = wid * B_PER_W

    pltpu.sync_copy(xT_hbm.at[:, pl.ds(col0, B_PER_W)], idx_v)

    def gather(s, buf, sem):
        pltpu.async_copy(tpad_hbm.at[idx_v.at[s]], buf, sem)

    def out_block(s):
        return out_hbm.at[s, :, pl.ds(col0, B_PER_W)]

    rows16 = lax.iota(jnp.int32, L)

    def transpose(g, o):
        for f in range(D):
            colf = jnp.full((L,), f, jnp.int32)
            for k in range(B_PER_W // L):
                v = plsc.load_gather(g, [rows16 + k * L, colf])
                o[f, pl.ds(k * L, L)] = v

    gather(0, G[0], gs[0])
    gather(1, G[1], gs[1])

    @pl.loop(0, SEQ, step=2)
    def _blk(i):
        for b in range(2):
            s = i + b
            pltpu.make_async_copy(tpad_hbm.at[idx_v.at[s]], G[b], gs[b]).wait()

            @pl.when(s >= 2)
            def _wait_out():
                pltpu.make_async_copy(O[b], out_block(s - 2), ss[b]).wait()

            transpose(G[b], O[b])

            @pl.when(s + 2 < SEQ)
            def _prefetch():
                gather(s + 2, G[b], gs[b])

            pltpu.async_copy(O[b], out_block(s), ss[b])

    for s in (SEQ - 2, SEQ - 1):
        b = s % 2
        pltpu.make_async_copy(O[b], out_block(s), ss[b]).wait()


@jax.jit
def _sc_gather(xT, tpad):
    mesh = plsc.VectorSubcoreMesh(core_axis_name="c", subcore_axis_name="s")
    return pl.kernel(
        _gather_body,
        out_type=jax.ShapeDtypeStruct((SEQ, D, BATCH), jnp.float32),
        mesh=mesh,
        scratch_types=(
            [pltpu.VMEM((SEQ, B_PER_W), jnp.int32)]
            + [pltpu.VMEM((B_PER_W, DP), jnp.float32) for _ in range(2)]
            + [pltpu.VMEM((D, B_PER_W), jnp.float32) for _ in range(2)]
            + [pltpu.SemaphoreType.DMA for _ in range(4)]
        ),
        compiler_params=pltpu.CompilerParams(use_tc_tiling_on_sc=True, needs_layout_passes=False),
    )(xT, tpad)


def kernel(x, table):
    tpad = jnp.pad(table, ((0, 0), (0, DP - D)))
    outT = _sc_gather(x.T, tpad)
    return jnp.transpose(outT, (2, 0, 1))


# speedup vs baseline: 1.9299x; 1.9299x over previous
import jax
import jax.numpy as jnp
from jax import lax
from jax.experimental import pallas as pl
from jax.experimental.pallas import tpu as pltpu
from jax.experimental.pallas import tpu_sc as plsc

VOCAB = 1000000
D = 64
DP = 128
BATCH = 4096
SEQ = 200
N = BATCH * SEQ

NC = 2
NS = 16
NW = NC * NS
B_PER_W = BATCH // NW
PER_W = B_PER_W * SEQ
NB = 4
AHEAD = 2


def _gather_body(xf_hbm, tpad_hbm, out_hbm, idx_v, r0, r1, r2, r3,
                 g0, g1, g2, g3, s0, s1, s2, s3):
    rows = [r0, r1, r2, r3]
    gs = [g0, g1, g2, g3]
    ss = [s0, s1, s2, s3]
    wid = lax.axis_index("s") * NC + lax.axis_index("c")
    b_base = wid * B_PER_W

    pltpu.sync_copy(xf_hbm.at[pl.ds(wid * PER_W, PER_W)], idx_v)

    def gather(j, buf, sem):
        pltpu.async_copy(tpad_hbm.at[idx_v.at[pl.ds(j * SEQ, SEQ)]], buf, sem)

    def out_slice(j):
        return out_hbm.at[b_base + j]

    for j in range(AHEAD):
        gather(j, rows[j % NB], gs[j % NB])

    @pl.loop(0, B_PER_W, step=NB)
    def _block(i):
        for bb in range(NB):
            j = i + bb
            ga = j + AHEAD
            gb = (bb + AHEAD) % NB

            @pl.when(ga < B_PER_W)
            def _issue():
                @pl.when(ga >= NB)
                def _wait_sc():
                    pltpu.make_async_copy(rows[gb], out_slice(ga - NB),
                                          ss[gb]).wait()
                gather(ga, rows[gb], gs[gb])

            pltpu.make_async_copy(tpad_hbm.at[idx_v.at[pl.ds(j * SEQ, SEQ)]],
                                  rows[bb], gs[bb]).wait()
            pltpu.async_copy(rows[bb], out_slice(j), ss[bb])

    for bb in range(NB):
        j = B_PER_W - NB + bb
        pltpu.make_async_copy(rows[j % NB], out_slice(j), ss[j % NB]).wait()


@jax.jit
def _sc_gather(xf, tpad):
    mesh = plsc.VectorSubcoreMesh(core_axis_name="c", subcore_axis_name="s")
    return pl.kernel(
        _gather_body,
        out_type=jax.ShapeDtypeStruct((BATCH, SEQ, DP), jnp.float32),
        mesh=mesh,
        scratch_types=(
            [pltpu.VMEM((PER_W,), jnp.int32)]
            + [pltpu.VMEM((SEQ, DP), jnp.float32) for _ in range(NB)]
            + [pltpu.SemaphoreType.DMA for _ in range(2 * NB)]
        ),
        compiler_params=pltpu.CompilerParams(use_tc_tiling_on_sc=True),
    )(xf, tpad)


def kernel(x, table):
    tpad = jnp.pad(table, ((0, 0), (0, DP - D)))
    out = _sc_gather(x.reshape(N), tpad)
    return out[:, :, :D]


# speedup vs baseline: 2.0901x; 1.0830x over previous
import jax
import jax.numpy as jnp
from jax import lax
from jax.experimental import pallas as pl
from jax.experimental.pallas import tpu as pltpu
from jax.experimental.pallas import tpu_sc as plsc

VOCAB = 1000000
D = 64
BATCH = 4096
SEQ = 200
N = BATCH * SEQ

NC = 2
NS = 16
NW = NC * NS
B_PER_W = BATCH // NW
PER_W = B_PER_W * SEQ


def _gather_body(xf_hbm, table_hbm, out_hbm, idx_v, r0, r1, g0, g1, s0, s1):
    rows = [r0, r1]
    gs = [g0, g1]
    ss = [s0, s1]
    wid = lax.axis_index("s") * NC + lax.axis_index("c")
    b_base = wid * B_PER_W

    pltpu.sync_copy(xf_hbm.at[pl.ds(wid * PER_W, PER_W)], idx_v)

    def issue(b, buf, sem):
        base = b * SEQ

        @pl.loop(0, SEQ - 8, step=16)
        def _j(j0):
            vec = idx_v[pl.ds(base + j0, 16)]
            for v in range(16):
                pltpu.async_copy(table_hbm.at[pl.ds(vec[v], 1)],
                                 buf.at[pl.ds(j0 + v, 1)], sem)

        vec = idx_v[pl.ds(base + SEQ - 16, 16)]
        for v in range(8, 16):
            pltpu.async_copy(table_hbm.at[pl.ds(vec[v], 1)],
                             buf.at[pl.ds(SEQ - 16 + v, 1)], sem)

    def drain_gather(buf, sem):
        pltpu.make_async_copy(table_hbm.at[pl.ds(0, SEQ)], buf, sem).wait()

    def out_slice(b):
        return out_hbm.at[b_base + b]

    issue(0, rows[0], gs[0])

    @pl.loop(0, B_PER_W, step=2)
    def _blk(i):
        for u in range(2):
            b = i + u

            @pl.when(b + 1 < B_PER_W)
            def _next():
                @pl.when(b >= 1)
                def _wait_sc():
                    pltpu.make_async_copy(rows[1 - u], out_slice(b - 1),
                                          ss[1 - u]).wait()
                issue(b + 1, rows[1 - u], gs[1 - u])

            drain_gather(rows[u], gs[u])
            pltpu.async_copy(rows[u], out_slice(b), ss[u])

    pltpu.make_async_copy(rows[0], out_slice(B_PER_W - 2), ss[0]).wait()
    pltpu.make_async_copy(rows[1], out_slice(B_PER_W - 1), ss[1]).wait()


@jax.jit
def _sc_gather(xf, table):
    mesh = plsc.VectorSubcoreMesh(core_axis_name="c", subcore_axis_name="s")
    return pl.kernel(
        _gather_body,
        out_type=jax.ShapeDtypeStruct((BATCH, SEQ, D), jnp.float32),
        mesh=mesh,
        scratch_types=(
            [pltpu.VMEM((PER_W,), jnp.int32)]
            + [pltpu.VMEM((SEQ, D), jnp.float32) for _ in range(2)]
            + [pltpu.SemaphoreType.DMA for _ in range(4)]
        ),
        compiler_params=pltpu.CompilerParams(use_tc_tiling_on_sc=True),
    )(xf, table)


def kernel(x, table):
    return _sc_gather(x.reshape(N), table)
